# baseline (speedup 1.0000x reference)
import functools

import jax
import jax.numpy as jnp
from jax import lax
from jax.experimental import pallas as pl
from jax.experimental.pallas import tpu as pltpu
from jax.experimental.pallas import tpu_sc as plsc

VOCAB = 1000000
DIM = 64
BATCH = 16384
HIST = 50

NW = 32
B = BATCH * HIST

NFULL = VOCAB // 128
CPW_A = NFULL // NW
XTRA_A = NFULL - CPW_A * NW
PACK = VOCAB // 2

C = 128
CPW_B = B // (NW * C)
NB = 4


def _pack_body(tabT_hbm, tail_hbm, scr_hbm, sl0, sl1, st0, st1,
               g0, g1, s0, s1):
    slabs = (sl0, sl1)
    stgs = (st0, st1)
    gsems = (g0, g1)
    ssems = (s0, s1)
    wid = lax.axis_index("c") * 16 + lax.axis_index("s")

    iota = lax.iota(jnp.int32, 16)

    def start_slab(c, b):
        pltpu.make_async_copy(
            tabT_hbm.at[:, pl.ds(c * 128, 128)], slabs[b], gsems[b]).start()

    def wait_slab(c, b):
        pltpu.make_async_copy(
            tabT_hbm.at[:, pl.ds(c * 128, 128)], slabs[b], gsems[b]).wait()

    def start_store(c, b):
        pltpu.make_async_copy(
            stgs[b], scr_hbm.at[pl.ds(c * 64, 64)], ssems[b]).start()

    def wait_store(c, b):
        pltpu.make_async_copy(
            stgs[b], scr_hbm.at[pl.ds(c * 64, 64)], ssems[b]).wait()

    def transpose(b):
        def prow(p, carry):
            for kk in range(8):
                dvec = 16 * (kk % 4) + iota
                lvec = jnp.full((16,), 2 * p + kk // 4, jnp.int32)
                v = plsc.load_gather(slabs[b], [dvec, lvec])
                stgs[b][p, pl.ds(16 * kk, 16)] = v
            return carry
        lax.fori_loop(0, 64, prow, 0)

    def col(j, b):
        return wid + NW * j if b == 0 else wid + NW * j + NW

    start_slab(col(0, 0), 0)
    start_slab(col(0, 1), 1)

    def body2(i, carry):
        j = 2 * i
        for b in range(2):
            c = col(j, b)
            wait_slab(c, b)

            @pl.when(i > 0)
            def _():
                wait_store(col(j - 2, b), b)

            transpose(b)

            @pl.when(i + 1 < CPW_A // 2)
            def _():
                start_slab(col(j + 2, b), b)

            start_store(c, b)
        return carry

    lax.fori_loop(0, CPW_A // 2, body2, 0)
    for b in range(2):
        wait_store(col(CPW_A - 2, b), b)

    @pl.when(wid < XTRA_A)
    def _():
        c = NFULL - XTRA_A + wid
        start_slab(c, 0)
        wait_slab(c, 0)
        transpose(0)
        start_store(c, 0)
        wait_store(c, 0)

    @pl.when(wid == NW - 1)
    def _():
        pltpu.make_async_copy(tail_hbm, sl0, g0).start()
        pltpu.make_async_copy(tail_hbm, sl0, g0).wait()
        pltpu.make_async_copy(
            sl0.at[pl.ds(0, 32)], scr_hbm.at[pl.ds(PACK - 32, 32)],
            s0).start()
        pltpu.make_async_copy(
            sl0.at[pl.ds(0, 32)], scr_hbm.at[pl.ds(PACK - 32, 32)],
            s0).wait()


def _emb_body(x_hbm, tab_hbm, out_hbm, idx_v, b0, b1, b2, b3,
              g0, g1, g2, g3, s0, s1, s2, s3):
    bufs = (b0, b1, b2, b3)
    gsems = (g0, g1, g2, g3)
    ssems = (s0, s1, s2, s3)
    wid = lax.axis_index("c") * 16 + lax.axis_index("s")
    base = wid * CPW_B

    pltpu.sync_copy(x_hbm.at[pl.ds(base, CPW_B)], idx_v)

    def start_g(j, b):
        pltpu.make_async_copy(
            tab_hbm.at[idx_v.at[j]], bufs[b], gsems[b]).start()

    def wait_g(j, b):
        pltpu.make_async_copy(
            tab_hbm.at[idx_v.at[j]], bufs[b], gsems[b]).wait()

    def start_s(j, b):
        pltpu.make_async_copy(bufs[b], out_hbm.at[base + j], ssems[b]).start()

    def wait_s(j, b):
        pltpu.make_async_copy(bufs[b], out_hbm.at[base + j], ssems[b]).wait()

    for b in range(NB):
        start_g(b, b)

    def body(i, carry):
        g = i * NB
        for b in range(NB):
            wait_g(g + b, b)
            start_s(g + b, b)
        for b in range(NB):
            wait_s(g + b, b)
            start_g(g + NB + b, b)
        return carry

    lax.fori_loop(0, CPW_B // NB - 1, body, 0)

    g = CPW_B - NB
    for b in range(NB):
        wait_g(g + b, b)
        start_s(g + b, b)
    for b in range(NB):
        wait_s(g + b, b)


@jax.jit
def _pipeline(x, W_in):
    mesh = plsc.VectorSubcoreMesh(core_axis_name="c", subcore_axis_name="s")

    pack = functools.partial(
        pl.kernel,
        mesh=mesh,
        out_type=jax.ShapeDtypeStruct((PACK, 128), jnp.float32),
        scratch_types=(
            [pltpu.VMEM((64, 128), jnp.float32) for _ in range(4)]
            + [pltpu.SemaphoreType.DMA for _ in range(4)]
        ),
        compiler_params=pltpu.CompilerParams(use_tc_tiling_on_sc=True,
                                             needs_layout_passes=False),
    )(_pack_body)

    emb = functools.partial(
        pl.kernel,
        mesh=mesh,
        out_type=jax.ShapeDtypeStruct((B // C, C, DIM), jnp.float32),
        scratch_types=(
            [pltpu.VMEM((CPW_B, C), jnp.int32)]
            + [pltpu.VMEM((C, DIM), jnp.float32) for _ in range(NB)]
            + [pltpu.SemaphoreType.DMA for _ in range(2 * NB)]
        ),
        compiler_params=pltpu.CompilerParams(use_tc_tiling_on_sc=False),
    )(_emb_body)

    tail = W_in[VOCAB - 64:].reshape(32, 128)
    tail = jnp.concatenate([tail, tail], axis=0)
    scr = pack(W_in.T, tail)
    tab_lin = scr.reshape(VOCAB, DIM)
    x2d = x.reshape(B // C, C).astype(jnp.int32)
    out = emb(x2d, tab_lin)
    return out.reshape(BATCH, HIST, DIM)


def kernel(x, W_in, W_out):
    return _pipeline(x, W_in)


# speedup vs baseline: 1.7099x; 1.7099x over previous
import functools

import jax
import jax.numpy as jnp
from jax import lax
from jax.experimental import pallas as pl
from jax.experimental.pallas import tpu as pltpu
from jax.experimental.pallas import tpu_sc as plsc

VOCAB = 1000000
DIM = 64
BATCH = 16384
HIST = 50

NW = 32
C = 128
B = BATCH * HIST
CPW = B // (NW * C)
NB = 4


def _emb_body(x_hbm, tab_hbm, out_hbm, idx_v, b0, b1, b2, b3,
              g0, g1, g2, g3, s0, s1, s2, s3):
    bufs = (b0, b1, b2, b3)
    gsems = (g0, g1, g2, g3)
    ssems = (s0, s1, s2, s3)
    wid = lax.axis_index("c") * 16 + lax.axis_index("s")
    base = wid * CPW

    pltpu.sync_copy(x_hbm.at[pl.ds(base, CPW)], idx_v)

    def start_g(j, b):
        pltpu.make_async_copy(
            tab_hbm.at[idx_v.at[j]], bufs[b], gsems[b]).start()

    def wait_g(j, b):
        pltpu.make_async_copy(
            tab_hbm.at[idx_v.at[j]], bufs[b], gsems[b]).wait()

    def start_s(j, b):
        pltpu.make_async_copy(bufs[b], out_hbm.at[base + j], ssems[b]).start()

    def wait_s(j, b):
        pltpu.make_async_copy(bufs[b], out_hbm.at[base + j], ssems[b]).wait()

    for b in range(NB):
        start_g(b, b)

    def body(i, carry):
        g = i * NB
        for b in range(NB):
            wait_g(g + b, b)
            start_s(g + b, b)
        for b in range(NB):
            wait_s(g + b, b)
            start_g(g + NB + b, b)
        return carry

    lax.fori_loop(0, CPW // NB - 1, body, 0)

    g = CPW - NB
    for b in range(NB):
        wait_g(g + b, b)
        start_s(g + b, b)
    for b in range(NB):
        wait_s(g + b, b)


@functools.partial(jax.jit, static_argnums=())
def _embed(x2d, table):
    mesh = plsc.VectorSubcoreMesh(core_axis_name="c", subcore_axis_name="s")
    f = functools.partial(
        pl.kernel,
        mesh=mesh,
        out_type=jax.ShapeDtypeStruct((B // C, C, DIM), jnp.float32),
        scratch_types=(
            [pltpu.VMEM((CPW, C), jnp.int32)]
            + [pltpu.VMEM((C, DIM), jnp.float32) for _ in range(NB)]
            + [pltpu.SemaphoreType.DMA for _ in range(2 * NB)]
        ),
        compiler_params=pltpu.CompilerParams(use_tc_tiling_on_sc=False),
    )(_emb_body)
    return f(x2d, table)


def kernel(x, W_in, W_out):
    x2d = x.reshape(B // C, C).astype(jnp.int32)
    out = _embed(x2d, W_in)
    return out.reshape(BATCH, HIST, DIM)
